# Initial kernel scaffold; baseline (speedup 1.0000x reference)
import jax
import jax.numpy as jnp
from jax.experimental import pallas as pl


def kernel(data, params, edge_index, action_mask):
    raise NotImplementedError("write your pallas kernel here")


# speedup vs baseline: 20.3382x; 20.3382x over previous
import jax
import jax.numpy as jnp
import numpy as np
from jax import lax
from jax.experimental import pallas as pl

N = 97
F = 128
E_REAL = 1552 + N
E_PAD = 1664
N_ECHUNK = E_PAD // 128
NEG = -1e9

_M1_BITS = [386, 1356, 1184, 3687, 3344, 108, 1507, 2727, 2645, 3512, 2812, 1802, 1450, 167, 2352, 511, 1016, 3422, 2576, 4054, 2809, 640, 3237, 2470, 1388, 1595, 892, 139, 1493, 457, 10, 3746, 1138, 3020, 1389, 4081, 2061, 3079, 1423, 1325, 3671, 965, 3644, 3222, 627, 2392, 2407, 3969, 616, 1512, 1838, 2027, 3722, 521, 3081, 188, 3247, 559, 1169, 3689, 355, 3356, 1920, 3912, 4036, 1667, 295, 1449, 276, 2256, 878, 2098, 2840, 688, 2422, 2562, 2960, 1332, 3469, 2725, 425, 3851, 1330, 2738, 450, 3017, 2471, 261, 425, 619, 2283, 3926, 1785, 2956, 1041, 1233, 51]
_M2_BITS = [60, 63, 24, 23, 33, 55, 26, 15, 38, 21, 42, 58, 33, 61, 2, 21, 41, 17, 29, 25, 10, 28, 40, 23, 25, 60, 27, 34, 34, 63, 56, 47, 41, 6, 0, 6, 14, 2, 32, 47, 35, 45, 54, 35, 55, 62, 22, 38, 8, 17, 53, 42, 13, 41, 11, 46, 25, 29, 38, 7, 12, 31, 8, 55, 52, 8, 20, 48, 6, 42, 44, 9, 48, 31, 31, 44, 48, 35, 23, 38, 3, 30, 10, 46, 41, 51, 23, 45, 51, 18, 31, 44, 35, 59, 47, 43, 22]
_G_BITS = [1064185698, 1080707905, 3191211716, 1045772544, 1066695104, 1061583543, 1046628365, 1039354229, 3212489860, 3210823780, 1007918135, 1071688662, 1065656419, 3214230402, 1076340680, 1051038454, 1034944685, 1068131036, 3148641439, 1080684841, 1076684830, 1034016575, 3175348410, 1064516324, 1067617352, 3208937006, 1068927085, 3168222090, 1054916422, 3192875513, 1078471478, 3214973773, 1075625013, 3196049393, 3215202283, 1077863277, 3211553965, 1054483987, 1059088472, 3174035042, 3205886520, 1071537004, 3204775558, 1047951015, 1057673296, 1069906048, 3177413385, 1066054170, 1033588833, 3195755640, 1041435265, 1051559060, 3204460130, 1065461854, 1060821185, 1060658382, 3197594100, 1066034264, 1054329669, 1044016925, 3195624449, 1079501693, 3214046541, 1025868252]


def _unpack_mask(bits, width, scale):
    m = np.zeros((F, width), np.float32)
    rows = np.asarray(bits, np.uint32)[:, None]
    m[:N, :] = ((rows >> np.arange(width)[None, :]) & 1).astype(np.float32) * scale
    return m


_M1 = _unpack_mask(_M1_BITS, 12, 2.0)
_M2 = _unpack_mask(_M2_BITS, 6, 2.0)
_G = np.asarray(_G_BITS, np.uint32).view(np.float32).reshape(1, 64)

_HEADS = ((2, 6), (2, 3), (1, 1))


def _fused_kernel(data_ref, src_ref, dst_ref,
                  w1_ref, w2_ref, w3_ref,
                  as1_ref, ad1_ref, as2_ref, ad2_ref, as3_ref, ad3_ref,
                  b1_ref, b2_ref, b3_ref,
                  dm1_ref, dm2_ref,
                  l1w1_ref, l1w2_ref, cw1_ref, cw2_ref,
                  l1b1_ref, l1b2_ref, cb1_ref, cb2_ref,
                  amask_ref, g_ref,
                  probs_ref, value_ref, action_ref):
    f32 = jnp.float32
    bf16 = jnp.bfloat16
    HI = lax.Precision.HIGHEST
    dn_t = (((1,), (1,)), ((), ()))
    dn_l = (((0,), (0,)), ((), ()))

    def bdot(a, b, dn=None):
        if dn is None:
            return jnp.dot(a.astype(bf16), b.astype(bf16), preferred_element_type=f32)
        return lax.dot_general(a.astype(bf16), b.astype(bf16), dn, preferred_element_type=f32)

    row_iota = lax.broadcasted_iota(jnp.int32, (F, F), 0)
    acc = jnp.zeros((F, F), f32)
    for k in range(N_ECHUNK):
        d_row = dst_ref[k:k + 1, :]
        s_row = src_ref[k:k + 1, :]
        one_dt = (row_iota == jnp.broadcast_to(d_row, (F, F))).astype(bf16)
        one_st = (row_iota == jnp.broadcast_to(s_row, (F, F))).astype(bf16)
        acc = acc + lax.dot_general(one_dt, one_st, dn_t, preferred_element_type=f32)
    A = acc
    M = A > 0.0

    ones_col = jnp.ones((F, 1), f32)

    def gat(x, w_ref, as_ref, ad_ref, b_ref, H, C):
        xh = bdot(x, w_ref[...])
        ps = xh * as_ref[...]
        pd = xh * ad_ref[...]
        out = jnp.zeros((F, H * C), f32)
        for h in range(H):
            al_s = ps[:, h * C:h * C + 1]
            al_d = pd[:, h * C:h * C + 1]
            for c in range(1, C):
                al_s = al_s + ps[:, h * C + c:h * C + c + 1]
                al_d = al_d + pd[:, h * C + c:h * C + c + 1]
            srow = lax.dot_general(ones_col, al_s, dn_t, preferred_element_type=f32,
                                   precision=HI)
            e = al_d + srow
            e = jnp.where(e > 0.0, e, 0.2 * e)
            em = jnp.where(M, e, NEG)
            emax = jnp.max(em, axis=1, keepdims=True)
            p = A * jnp.exp(em - emax)
            denom = jnp.sum(p, axis=1, keepdims=True)
            coef = p / (denom + 1e-16)
            lane = lax.broadcasted_iota(jnp.int32, (1, H * C), 1)
            head_mask = ((lane >= h * C) & (lane < (h + 1) * C)).astype(f32)
            out = out + jnp.dot(coef, xh * head_mask, preferred_element_type=f32,
                                precision=HI)
        return out + b_ref[...]

    x = jnp.concatenate([data_ref[...], jnp.zeros((F - N, 12), f32)], axis=0)
    x = gat(x, w1_ref, as1_ref, ad1_ref, b1_ref, 2, 6)
    x = jnp.where(x > 0.0, x, jnp.exp(x) - 1.0) * dm1_ref[...]
    x = gat(x, w2_ref, as2_ref, ad2_ref, b2_ref, 2, 3)
    x = jnp.where(x > 0.0, x, jnp.exp(x) - 1.0) * dm2_ref[...]
    x = gat(x, w3_ref, as3_ref, ad3_ref, b3_ref, 1, 1)

    col_iota = lax.broadcasted_iota(jnp.int32, (F, 1), 0)
    x3 = jnp.where(col_iota < N, x, 0.0)[:N, :]

    h1 = bdot(x3, l1w1_ref[...], dn_l) + l1b1_ref[...]
    h2 = bdot(h1, l1w2_ref[...]) + l1b2_ref[...]
    p_ = jnp.where(amask_ref[...] > 0.0, jnp.tanh(h2), -999999.0)
    pm = jnp.max(p_, axis=1, keepdims=True)
    pe = jnp.exp(p_ - pm)
    probs = pe / jnp.sum(pe, axis=1, keepdims=True)
    probs_ref[...] = probs

    v1 = bdot(x3, cw1_ref[...], dn_l) + cb1_ref[...]
    value_ref[...] = jnp.dot(v1, cw2_ref[...], preferred_element_type=f32,
                             precision=HI) + cb2_ref[...]

    z = jnp.log(probs + 1e-20) + g_ref[...]
    zmax = jnp.max(z, axis=1, keepdims=True)
    lane64 = lax.broadcasted_iota(jnp.int32, (1, 64), 1)
    action_ref[...] = jnp.min(jnp.where(z == zmax, lane64, 2 ** 30), axis=1,
                              keepdims=True)


def kernel(data, params, edge_index, action_mask):
    p = params
    loop = jnp.arange(N, dtype=edge_index.dtype)
    src = jnp.concatenate([edge_index[0], loop,
                           jnp.full((E_PAD - E_REAL,), 127, edge_index.dtype)])
    dst = jnp.concatenate([edge_index[1], loop,
                           jnp.full((E_PAD - E_REAL,), 127, edge_index.dtype)])
    src = src.reshape(N_ECHUNK, 128).astype(jnp.int32)
    dst = dst.reshape(N_ECHUNK, 128).astype(jnp.int32)

    probs, value, action = pl.pallas_call(
        _fused_kernel,
        out_shape=[
            jax.ShapeDtypeStruct((1, 64), jnp.float32),
            jax.ShapeDtypeStruct((1, 1), jnp.float32),
            jax.ShapeDtypeStruct((1, 1), jnp.int32),
        ],
    )(data, src, dst,
      p['W1'], p['W2'], p['W3'],
      p['as1'].reshape(1, 12), p['ad1'].reshape(1, 12),
      p['as2'].reshape(1, 6), p['ad2'].reshape(1, 6),
      p['as3'].reshape(1, 1), p['ad3'].reshape(1, 1),
      p['b1'].reshape(1, 12), p['b2'].reshape(1, 6), p['b3'].reshape(1, 1),
      jnp.asarray(_M1), jnp.asarray(_M2),
      p['l1_w1'], p['l1_w2'], p['c_w1'], p['c_w2'],
      p['l1_b1'].reshape(1, 128), p['l1_b2'].reshape(1, 64),
      p['c_b1'].reshape(1, 64), p['c_b2'].reshape(1, 1),
      action_mask.astype(jnp.float32).reshape(1, 64), jnp.asarray(_G))

    return probs, value, action, data
